# baseline (speedup 1.0000x reference)
import functools

import jax
import jax.numpy as jnp
from jax import lax
from jax.experimental import pallas as pl
from jax.experimental.pallas import tpu as pltpu
from jax.experimental.pallas import tpu_sc as plsc

_R = 64
_D = 128

_NC = 2
_NS = 16
_NW = _NC * _NS

_CH = 128
_NBUF = 4
_LOOK = 3


def _rne16(x):
    t = jax.lax.bitcast_convert_type(x, jnp.int32)
    return (t + jnp.int32(0x7FFF) + ((t >> 16) & jnp.int32(1))) >> 16


_BLK = 16384


def _matmul_body(ut_ref, b_ref, w_ref):
    w = jax.lax.dot_general(
        ut_ref[...], b_ref[...],
        dimension_numbers=(((0,), (0,)), ((), ())),
        preferred_element_type=jnp.float32,
    )
    lo = _rne16(w[:, : _D // 2]) & jnp.int32(0xFFFF)
    hi = _rne16(w[:, _D // 2:]) << 16
    z = lo | hi
    h = z.shape[0] // 2
    w_ref[...] = jnp.concatenate([z[:h, :], z[h:, :]], axis=1)


def _compute_w_packed(U, B):
    m = U.shape[0]
    grid = (m + _BLK - 1) // _BLK
    return pl.pallas_call(
        _matmul_body,
        grid=(grid,),
        in_specs=[
            pl.BlockSpec((_R, _BLK), lambda i: (0, i)),
            pl.BlockSpec((_R, _D), lambda i: (0, 0)),
        ],
        out_specs=pl.BlockSpec((_BLK // 2, _D), lambda i: (i, 0)),
        out_shape=jax.ShapeDtypeStruct((grid * _BLK // 2, _D), jnp.int32),
    )(U.T, B)


def _make_gather(ntot, nch):
    mesh = plsc.VectorSubcoreMesh(core_axis_name="c", subcore_axis_name="s")
    per_w = nch * _CH

    @functools.partial(
        pl.kernel,
        out_type=jax.ShapeDtypeStruct((ntot, _D), jnp.int32),
        mesh=mesh,
        compiler_params=pltpu.CompilerParams(use_tc_tiling_on_sc=False),
        scratch_types=[
            pltpu.VMEM((per_w,), jnp.int32),
            [pltpu.VMEM((_CH, _D // 2), jnp.int32)] * _NBUF,
            [pltpu.VMEM((_CH, _D), jnp.int32)] * _NBUF,
            [pltpu.SemaphoreType.DMA] * _NBUF,
            [pltpu.SemaphoreType.DMA] * _NBUF,
        ],
    )
    def gather(table_hbm, idx_hbm, out_hbm, idx_v, braw, rows, gsems, osems):
        wid = lax.axis_index("s") * _NC + lax.axis_index("c")
        base = wid * per_w
        pltpu.sync_copy(idx_hbm.at[pl.ds(base, per_w)], idx_v)

        def gather_chunk(c, buf):
            pltpu.make_async_copy(
                table_hbm.at[idx_v.at[pl.ds(c * _CH, _CH)]],
                braw[buf],
                gsems[buf],
            ).start()

        def out_copy(c, buf):
            return pltpu.make_async_copy(
                rows[buf],
                out_hbm.at[pl.ds(base + c * _CH, _CH)],
                osems[buf],
            )

        def widen(buf):
            bsrc = braw[buf]
            fdst = rows[buf]

            def per_row(r, _):
                for k in range(4):
                    xi = bsrc[r, pl.ds(16 * k, 16)]
                    fdst[r, pl.ds(16 * k, 16)] = xi << 16
                    fdst[r, pl.ds(64 + 16 * k, 16)] = xi & jnp.int32(-65536)
                return 0

            lax.fori_loop(0, _CH, per_row, 0, unroll=4)

        for c in range(_LOOK):
            gather_chunk(c, c % _NBUF)

        def body(g, _):
            for b in range(_NBUF):
                c = _NBUF * g + b
                nxt = c + _LOOK
                nbuf = (b + _LOOK) % _NBUF

                @pl.when(nxt < nch)
                def _():
                    @pl.when(c >= 1)
                    def _():
                        out_copy(c - 1, nbuf).wait()

                    gather_chunk(nxt, nbuf)

                pltpu.make_async_copy(
                    table_hbm.at[idx_v.at[pl.ds(c * _CH, _CH)]],
                    braw[b],
                    gsems[b],
                ).wait()
                widen(b)
                out_copy(c, b).start()
            return 0

        lax.fori_loop(0, nch // _NBUF, body, 0, unroll=False)
        for k in range(nch - _NBUF, nch):
            out_copy(k, k % _NBUF).wait()

    return gather


def kernel(local_ids, U, B):
    bsz, seq = local_ids.shape
    ntot = bsz * seq
    nch = ntot // (_NW * _CH)

    Wp = _compute_w_packed(U, B)
    Wp = Wp.reshape(Wp.shape[0] * 2, _D // 2)
    ids = local_ids.astype(jnp.int32).reshape(ntot)
    ids = ((ids & ~(_BLK - 1))
           + ((ids & (_BLK // 2 - 1)) << 1)
           + ((ids & (_BLK - 1)) >> 13))
    out = _make_gather(ntot, nch)(Wp, ids)
    return jax.lax.bitcast_convert_type(out, jnp.float32).reshape(bsz, seq, _D)


# speedup vs baseline: 1.6126x; 1.6126x over previous
import functools

import jax
import jax.numpy as jnp
from jax import lax
from jax.experimental import pallas as pl
from jax.experimental.pallas import tpu as pltpu
from jax.experimental.pallas import tpu_sc as plsc

_R = 64
_D = 128

_NC = 2
_NS = 16
_NW = _NC * _NS

_CH = 128
_NBUF = 4
_LOOK = 3


def _rne16(x):
    t = jax.lax.bitcast_convert_type(x, jnp.int32)
    return (t + jnp.int32(0x7FFF) + ((t >> 16) & jnp.int32(1))) >> 16


_BLK = 16384


def _matmul_body(ut_ref, b_ref, w_ref):
    w = jax.lax.dot_general(
        ut_ref[...], b_ref[...],
        dimension_numbers=(((0,), (0,)), ((), ())),
        preferred_element_type=jnp.float32,
    )
    lo = _rne16(w[:, : _D // 2]) & jnp.int32(0xFFFF)
    hi = _rne16(w[:, _D // 2:]) << 16
    z = lo | hi
    h = z.shape[0] // 2
    w_ref[...] = jnp.concatenate([z[:h, :], z[h:, :]], axis=1)


def _compute_w_packed(U, B):
    m = U.shape[0]
    grid = (m + _BLK - 1) // _BLK
    return pl.pallas_call(
        _matmul_body,
        grid=(grid,),
        in_specs=[
            pl.BlockSpec((_R, _BLK), lambda i: (0, i)),
            pl.BlockSpec((_R, _D), lambda i: (0, 0)),
        ],
        out_specs=pl.BlockSpec((_BLK // 2, _D), lambda i: (i, 0)),
        out_shape=jax.ShapeDtypeStruct((grid * _BLK // 2, _D), jnp.int32),
    )(U.T, B)


def _make_gather(ntot, nch):
    mesh = plsc.VectorSubcoreMesh(core_axis_name="c", subcore_axis_name="s")
    per_w = nch * _CH

    @functools.partial(
        pl.kernel,
        out_type=jax.ShapeDtypeStruct((ntot, _D), jnp.int32),
        mesh=mesh,
        compiler_params=pltpu.CompilerParams(use_tc_tiling_on_sc=False),
        scratch_types=[
            pltpu.VMEM((per_w,), jnp.int32),
            [pltpu.VMEM((_CH, _D // 2), jnp.int32)] * _NBUF,
            [pltpu.VMEM((_CH, _D), jnp.int32)] * _NBUF,
            [pltpu.SemaphoreType.DMA] * _NBUF,
            [pltpu.SemaphoreType.DMA] * _NBUF,
        ],
    )
    def gather(table_hbm, idx_hbm, out_hbm, idx_v, braw, rows, gsems, osems):
        wid = lax.axis_index("s") * _NC + lax.axis_index("c")
        base = wid * per_w
        pltpu.sync_copy(idx_hbm.at[pl.ds(base, per_w)], idx_v)

        def gather_chunk(c, buf):
            pltpu.make_async_copy(
                table_hbm.at[idx_v.at[pl.ds(c * _CH, _CH)]],
                braw[buf],
                gsems[buf],
            ).start()

        def out_copy(c, buf):
            return pltpu.make_async_copy(
                rows[buf],
                out_hbm.at[pl.ds(base + c * _CH, _CH)],
                osems[buf],
            )

        def widen(buf):
            bsrc = braw[buf]
            fdst = rows[buf]

            @plsc.parallel_loop(0, _CH, 1, unroll=8)
            def per_row(r):
                for k in range(4):
                    xi = bsrc[r, pl.ds(16 * k, 16)]
                    fdst[r, pl.ds(16 * k, 16)] = xi << 16
                    fdst[r, pl.ds(64 + 16 * k, 16)] = xi & jnp.int32(-65536)

        for c in range(_LOOK):
            gather_chunk(c, c % _NBUF)

        def body(g, _):
            for b in range(_NBUF):
                c = _NBUF * g + b
                nxt = c + _LOOK
                nbuf = (b + _LOOK) % _NBUF

                @pl.when(nxt < nch)
                def _():
                    @pl.when(c >= 1)
                    def _():
                        out_copy(c - 1, nbuf).wait()

                    gather_chunk(nxt, nbuf)

                pltpu.make_async_copy(
                    table_hbm.at[idx_v.at[pl.ds(c * _CH, _CH)]],
                    braw[b],
                    gsems[b],
                ).wait()
                widen(b)
                out_copy(c, b).start()
            return 0

        lax.fori_loop(0, nch // _NBUF, body, 0, unroll=False)
        for k in range(nch - _NBUF, nch):
            out_copy(k, k % _NBUF).wait()

    return gather


def kernel(local_ids, U, B):
    bsz, seq = local_ids.shape
    ntot = bsz * seq
    nch = ntot // (_NW * _CH)

    Wp = _compute_w_packed(U, B)
    Wp = Wp.reshape(Wp.shape[0] * 2, _D // 2)
    ids = local_ids.astype(jnp.int32).reshape(ntot)
    ids = ((ids & ~(_BLK - 1))
           + ((ids & (_BLK // 2 - 1)) << 1)
           + ((ids & (_BLK - 1)) >> 13))
    out = _make_gather(ntot, nch)(Wp, ids)
    return jax.lax.bitcast_convert_type(out, jnp.float32).reshape(bsz, seq, _D)


# speedup vs baseline: 2.6476x; 1.6419x over previous
import functools

import jax
import jax.numpy as jnp
from jax import lax
from jax.experimental import pallas as pl
from jax.experimental.pallas import tpu as pltpu
from jax.experimental.pallas import tpu_sc as plsc

_R = 64
_D = 128

_NC = 2
_NS = 16
_NW = _NC * _NS

_CH = 128
_NBUF = 4
_LOOK = 3


def _matmul_body(ut_ref, b_ref, w_ref):
    w_ref[...] = jax.lax.dot_general(
        ut_ref[...], b_ref[...],
        dimension_numbers=(((0,), (0,)), ((), ())),
        preferred_element_type=jnp.float32,
    )


def _compute_w(U, B):
    m = U.shape[0]
    blk = 32768
    grid = (m + blk - 1) // blk
    return pl.pallas_call(
        _matmul_body,
        grid=(grid,),
        in_specs=[
            pl.BlockSpec((_R, blk), lambda i: (0, i)),
            pl.BlockSpec((_R, _D), lambda i: (0, 0)),
        ],
        out_specs=pl.BlockSpec((blk, _D), lambda i: (i, 0)),
        out_shape=jax.ShapeDtypeStruct((m, _D), jnp.float32),
    )(U.T, B)


def _make_gather(ntot, nch):
    mesh = plsc.VectorSubcoreMesh(core_axis_name="c", subcore_axis_name="s")
    per_w = nch * _CH

    @functools.partial(
        pl.kernel,
        out_type=jax.ShapeDtypeStruct((ntot, _D), jnp.float32),
        mesh=mesh,
        scratch_types=[
            pltpu.VMEM((per_w,), jnp.int32),
            [pltpu.VMEM((_CH, _D), jnp.float32)] * _NBUF,
            [pltpu.SemaphoreType.DMA] * _NBUF,
            [pltpu.SemaphoreType.DMA] * _NBUF,
        ],
    )
    def gather(table_hbm, idx_hbm, out_hbm, idx_v, rows, gsems, osems):
        wid = lax.axis_index("s") * _NC + lax.axis_index("c")
        base = wid * per_w
        pltpu.sync_copy(idx_hbm.at[pl.ds(base, per_w)], idx_v)

        def gather_chunk(c, buf):
            pltpu.make_async_copy(
                table_hbm.at[idx_v.at[pl.ds(c * _CH, _CH)]],
                rows[buf],
                gsems[buf],
            ).start()

        def out_copy(c, buf):
            return pltpu.make_async_copy(
                rows[buf],
                out_hbm.at[pl.ds(base + c * _CH, _CH)],
                osems[buf],
            )

        for c in range(_LOOK):
            gather_chunk(c, c % _NBUF)

        def body(g, _):
            for b in range(_NBUF):
                c = 4 * g + b
                nxt = c + _LOOK
                nbuf = (b + _LOOK) % _NBUF

                @pl.when(nxt < nch)
                def _():
                    @pl.when(c >= 1)
                    def _():
                        out_copy(c - 1, nbuf).wait()

                    gather_chunk(nxt, nbuf)

                pltpu.make_async_copy(
                    table_hbm.at[idx_v.at[pl.ds(c * _CH, _CH)]],
                    rows[b],
                    gsems[b],
                ).wait()
                out_copy(c, b).start()
            return 0

        lax.fori_loop(0, nch // _NBUF, body, 0, unroll=False)
        for k in range(nch - _NBUF, nch):
            out_copy(k, k % _NBUF).wait()

    return gather


def kernel(local_ids, U, B):
    bsz, seq = local_ids.shape
    ntot = bsz * seq
    nch = ntot // (_NW * _CH)

    W = _compute_w(U, B)
    ids = local_ids.astype(jnp.int32).reshape(ntot)
    out = _make_gather(ntot, nch)(W, ids)
    return out.reshape(bsz, seq, _D)
